# baseline (speedup 1.0000x reference)
import functools

import jax
import jax.numpy as jnp
from jax import lax
from jax.experimental import pallas as pl
from jax.experimental.pallas import tpu as pltpu
from jax.experimental.pallas import tpu_sc as plsc

BATCH = 16384
EMBED_DIM = 64
SUBROWS = 8
LANES = 16
NUM_CORES = 2
NUM_SUBCORES = 16
NUM_WORKERS = NUM_CORES * NUM_SUBCORES
B_PER_W = BATCH // NUM_WORKERS
CHUNK = LANES
CHUNKS_PER_W = B_PER_W // CHUNK

_PARAMS = pltpu.CompilerParams(needs_layout_passes=False,
                               use_tc_tiling_on_sc=True)


def _mf_body(uid_hbm, iid_hbm, ut_hbm, it_hbm, ubg_hbm, ibg_hbm, gb_hbm,
             out_hbm,
             uidx, iidx, ubuf, ibuf, ubias, ibias, gbv,
             outv, sem, semb):
    wid = lax.axis_index("s") * NUM_CORES + lax.axis_index("c")
    base = wid * B_PER_W

    pltpu.sync_copy(uid_hbm.at[pl.ds(base, B_PER_W)], uidx)
    pltpu.sync_copy(iid_hbm.at[pl.ds(base, B_PER_W)], iidx)

    cub = pltpu.async_copy(ubg_hbm.at[pl.ds(base, B_PER_W)], ubias, semb)
    cib = pltpu.async_copy(ibg_hbm.at[pl.ds(base, B_PER_W)], ibias, semb)
    pltpu.sync_copy(gb_hbm, gbv)
    cub.wait()
    cib.wait()

    gb = gbv[...]
    lane = lax.iota(jnp.int32, LANES)
    seven = jnp.full((LANES,), 7, jnp.int32)

    def chunk_step(c, _):
        cbase = c * CHUNK
        sl = pl.ds(cbase, LANES)
        idu = uidx[sl]
        idi = iidx[sl]
        copies = []
        for k in range(CHUNK):
            mk = lane == k
            su = jnp.max(jnp.where(mk, idu, 0), axis=0)
            si = jnp.max(jnp.where(mk, idi, 0), axis=0)
            tu = pl.multiple_of(su - lax.bitwise_and(su, 7), SUBROWS)
            ti = pl.multiple_of(si - lax.bitwise_and(si, 7), SUBROWS)
            copies.append(pltpu.async_copy(
                ut_hbm.at[pl.ds(tu, SUBROWS)], ubuf.at[k], sem))
            copies.append(pltpu.async_copy(
                it_hbm.at[pl.ds(ti, SUBROWS)], ibuf.at[k], sem))
        for cp in copies:
            cp.wait()

        usub = lax.bitwise_and(idu, seven)
        isub = lax.bitwise_and(idi, seven)
        acc = gb + ubias[sl] + ibias[sl]
        for d in range(EMBED_DIM):
            col = jnp.full((LANES,), d, jnp.int32)
            uv = plsc.load_gather(ubuf, [lane, usub, col])
            iv = plsc.load_gather(ibuf, [lane, isub, col])
            acc = acc + uv * iv
        outv[sl] = acc
        return 0

    lax.fori_loop(0, CHUNKS_PER_W, chunk_step, 0)

    pltpu.sync_copy(outv, out_hbm.at[pl.ds(base, B_PER_W)])


@jax.jit
def kernel(user_ids, item_ids, user_table, item_table, user_bias_table,
           item_bias_table, global_bias):
    uid32 = user_ids.astype(jnp.int32)
    iid32 = item_ids.astype(jnp.int32)
    gb16 = jnp.broadcast_to(global_bias, (LANES,))
    ubg = jnp.take(user_bias_table, uid32, axis=0).reshape(BATCH)
    ibg = jnp.take(item_bias_table, iid32, axis=0).reshape(BATCH)

    mesh = plsc.VectorSubcoreMesh(core_axis_name="c", subcore_axis_name="s",
                                  num_cores=NUM_CORES,
                                  num_subcores=NUM_SUBCORES)
    mf = pl.kernel(
        _mf_body,
        out_type=jax.ShapeDtypeStruct((BATCH,), jnp.float32),
        mesh=mesh,
        scratch_types=[
            pltpu.VMEM((B_PER_W,), jnp.int32),
            pltpu.VMEM((B_PER_W,), jnp.int32),
            pltpu.VMEM((CHUNK, SUBROWS, EMBED_DIM), jnp.float32),
            pltpu.VMEM((CHUNK, SUBROWS, EMBED_DIM), jnp.float32),
            pltpu.VMEM((B_PER_W,), jnp.float32),
            pltpu.VMEM((B_PER_W,), jnp.float32),
            pltpu.VMEM((LANES,), jnp.float32),
            pltpu.VMEM((B_PER_W,), jnp.float32),
            pltpu.SemaphoreType.DMA,
            pltpu.SemaphoreType.DMA,
        ],
        compiler_params=_PARAMS,
    )
    return mf(uid32, iid32, user_table, item_table, ubg, ibg, gb16)
